# Initial kernel scaffold; baseline (speedup 1.0000x reference)
import jax
import jax.numpy as jnp
from jax.experimental import pallas as pl


def kernel(x, edge_index, batch, W1, b1, W2, b2, W3, b3, Wlin, blin):
    raise NotImplementedError("write your pallas kernel here")


# speedup vs baseline: 9.3373x; 9.3373x over previous
import functools

import jax
import jax.numpy as jnp
from jax import lax
from jax.experimental import pallas as pl
from jax.experimental.pallas import tpu as pltpu
from jax.experimental.pallas import tpu_sc as plsc

N = 100000
E = 1600000
F_IN = 33
H = 64
C = 2
G = 16

EROWS = 12800
EPAD = EROWS * 128
DUMMY = N
SLAB = 102400
NBK = 2000
GRID = N // NBK


def _fill(ref, value):
    @pl.loop(0, 128)
    def _(i):
        ref[i, :] = jnp.full((16,), value, jnp.float32)


def _zero_slab(slab, zv, s):
    @pl.loop(0, 50)
    def _(k):
        pltpu.sync_copy(zv, slab.at[pl.ds(s * 6400 + k * 128, 128)])


def _writeback(slab, out, s):
    @pl.when(s < 15)
    def _():
        row = s * 6256
        pltpu.sync_copy(slab.at[pl.ds(row, 6256)], out.at[pl.ds(row, 6256)])

    @pl.when(s == 15)
    def _():
        pltpu.sync_copy(slab.at[pl.ds(93840, 6160)], out.at[pl.ds(93840, 6160)])


def _deg_body(dst_hbm, dega, degb, slab, dbuf, ones_v, zv, sem):
    c = lax.axis_index("c")
    s = lax.axis_index("s")
    _fill(ones_v, 1.0)
    _fill(zv, 0.0)
    _zero_slab(slab, zv, s)
    plsc.subcore_barrier()
    base = c * 6400 + s * 400

    @pl.loop(0, 50)
    def _(b):
        pltpu.sync_copy(dst_hbm.at[pl.ds(base + b * 8, 8)], dbuf)
        descs = [
            pltpu.async_copy(ones_v, slab.at[dbuf.at[j]], sem, add=True)
            for j in range(8)
        ]
        for d in descs:
            d.wait()

    plsc.subcore_barrier()

    @pl.when(c == 0)
    def _():
        _writeback(slab, dega, s)

    @pl.when(c == 1)
    def _():
        _writeback(slab, degb, s)


def _one_pass(hs, agg, src_hbm, dst_hbm, s, slab, sbuf, dbuf, rows, zv,
              gsem, ssem):
    _zero_slab(slab, zv, s)
    plsc.subcore_barrier()
    base = s * 800

    @pl.loop(0, 100)
    def _(b):
        row0 = base + b * 8
        pltpu.sync_copy(src_hbm.at[pl.ds(row0, 8)], sbuf)
        pltpu.sync_copy(dst_hbm.at[pl.ds(row0, 8)], dbuf)
        g = [
            pltpu.async_copy(hs.at[sbuf.at[j]],
                             rows.at[pl.ds(j * 128, 128)], gsem)
            for j in range(8)
        ]
        for d in g:
            d.wait()
        sc = [
            pltpu.async_copy(rows.at[pl.ds(j * 128, 128)],
                             slab.at[dbuf.at[j]], ssem, add=True)
            for j in range(8)
        ]
        for d in sc:
            d.wait()

    plsc.subcore_barrier()
    _writeback(slab, agg, s)


def _agg_body(h0, h1, h2, h3, src_hbm, dst_hbm, a0, a1, a2, a3,
              slab, sbuf, dbuf, rows, zv, gsem, ssem):
    c = lax.axis_index("c")
    s = lax.axis_index("s")
    _fill(zv, 0.0)
    hs_list = [h0, h1, h2, h3]
    out_list = [a0, a1, a2, a3]
    for q in range(4):
        @pl.when(c == q // 2)
        def _(q=q):
            _one_pass(hs_list[q], out_list[q], src_hbm, dst_hbm, s, slab,
                      sbuf, dbuf, rows, zv, gsem, ssem)


@functools.cache
def _sc_calls():
    mesh = plsc.VectorSubcoreMesh(core_axis_name="c", subcore_axis_name="s")
    deg_call = pl.kernel(
        _deg_body,
        out_type=(jax.ShapeDtypeStruct((N, 16), jnp.float32),
                  jax.ShapeDtypeStruct((N, 16), jnp.float32)),
        mesh=mesh,
        compiler_params=pltpu.CompilerParams(use_tc_tiling_on_sc=False),
        scratch_types=(
            pltpu.VMEM_SHARED((SLAB, 16), jnp.float32),
            pltpu.VMEM((8, 128), jnp.int32),
            pltpu.VMEM((128, 16), jnp.float32),
            pltpu.VMEM((128, 16), jnp.float32),
            pltpu.SemaphoreType.DMA,
        ),
    )
    agg_call = pl.kernel(
        _agg_body,
        out_type=tuple(jax.ShapeDtypeStruct((N, 16), jnp.float32)
                       for _ in range(4)),
        mesh=mesh,
        compiler_params=pltpu.CompilerParams(use_tc_tiling_on_sc=False),
        scratch_types=(
            pltpu.VMEM_SHARED((SLAB, 16), jnp.float32),
            pltpu.VMEM((8, 128), jnp.int32),
            pltpu.VMEM((8, 128), jnp.int32),
            pltpu.VMEM((1024, 16), jnp.float32),
            pltpu.VMEM((128, 16), jnp.float32),
            pltpu.SemaphoreType.DMA,
            pltpu.SemaphoreType.DMA,
        ),
    )
    return deg_call, agg_call


def _dinv_blk(da, db):
    return lax.rsqrt(1.0 + da[:, 0:1] + db[:, 0:1])


def _t1_body(x_ref, w_ref, da, db, o0, o1, o2, o3):
    dinv = _dinv_blk(da, db)
    t = jnp.dot(x_ref[...], w_ref[...],
                preferred_element_type=jnp.float32) * dinv
    o0[...] = t[:, 0:16]
    o1[...] = t[:, 16:32]
    o2[...] = t[:, 32:48]
    o3[...] = t[:, 48:64]


def _mid_body(a0, a1, a2, a3, h0, h1, h2, h3, da, db, b_ref, w_ref,
              o0, o1, o2, o3):
    dinv = _dinv_blk(da, db)
    agg = jnp.concatenate([a0[...], a1[...], a2[...], a3[...]], axis=1)
    hsb = jnp.concatenate([h0[...], h1[...], h2[...], h3[...]], axis=1)
    h = jnp.maximum(dinv * (agg + hsb) + b_ref[...], 0.0)
    t = jnp.dot(h, w_ref[...], preferred_element_type=jnp.float32) * dinv
    o0[...] = t[:, 0:16]
    o1[...] = t[:, 16:32]
    o2[...] = t[:, 32:48]
    o3[...] = t[:, 48:64]


def _t4_body(a0, a1, a2, a3, h0, h1, h2, h3, da, db, b_ref, batch_ref,
             wlin_ref, blin_ref, out_ref, pooled, cnt):
    i = pl.program_id(0)

    @pl.when(i == 0)
    def _():
        pooled[...] = jnp.zeros((G, H), jnp.float32)
        cnt[...] = jnp.zeros((G, 128), jnp.float32)

    dinv = _dinv_blk(da, db)
    agg = jnp.concatenate([a0[...], a1[...], a2[...], a3[...]], axis=1)
    hsb = jnp.concatenate([h0[...], h1[...], h2[...], h3[...]], axis=1)
    h3v = dinv * (agg + hsb) + b_ref[...]
    bb = batch_ref[0, 0, :]
    gi = lax.broadcasted_iota(jnp.int32, (G, NBK), 0)
    oh = (bb[None, :] == gi).astype(jnp.float32)
    pooled[...] += jnp.dot(oh, h3v, preferred_element_type=jnp.float32)
    cnt[...] = cnt[...] + jnp.sum(oh, axis=1, keepdims=True)

    @pl.when(i == GRID - 1)
    def _():
        p = pooled[...] / jnp.maximum(cnt[:, 0:1], 1.0)
        out_ref[...] = (jnp.dot(p, wlin_ref[...],
                                preferred_element_type=jnp.float32)
                        + blin_ref[...])


def _row_spec(w):
    return pl.BlockSpec((NBK, w), lambda i: (i, 0))


def _full_spec(shape):
    nd = len(shape)
    return pl.BlockSpec(shape, lambda i: (0,) * nd)


_t1_call = pl.pallas_call(
    _t1_body,
    grid=(GRID,),
    in_specs=[
        _row_spec(F_IN),
        _full_spec((F_IN, H)),
        _row_spec(16),
        _row_spec(16),
    ],
    out_specs=[_row_spec(16)] * 4,
    out_shape=[jax.ShapeDtypeStruct((N, 16), jnp.float32)] * 4,
)

_mid_call = pl.pallas_call(
    _mid_body,
    grid=(GRID,),
    in_specs=[_row_spec(16)] * 8 + [
        _row_spec(16),
        _row_spec(16),
        _full_spec((1, H)),
        _full_spec((H, H)),
    ],
    out_specs=[_row_spec(16)] * 4,
    out_shape=[jax.ShapeDtypeStruct((N, 16), jnp.float32)] * 4,
)

_t4_call = pl.pallas_call(
    _t4_body,
    grid=(GRID,),
    in_specs=[_row_spec(16)] * 8 + [
        _row_spec(16),
        _row_spec(16),
        _full_spec((1, H)),
        pl.BlockSpec((1, 1, NBK), lambda i: (i, 0, 0)),
        _full_spec((H, C)),
        _full_spec((1, C)),
    ],
    out_specs=pl.BlockSpec((G, C), lambda i: (0, 0)),
    out_shape=jax.ShapeDtypeStruct((G, C), jnp.float32),
    scratch_shapes=[
        pltpu.VMEM((G, H), jnp.float32),
        pltpu.VMEM((G, 128), jnp.float32),
    ],
)


@jax.jit
def kernel(x, edge_index, batch, W1, b1, W2, b2, W3, b3, Wlin, blin):
    pad = EPAD - E
    src_r = jnp.concatenate(
        [edge_index[0], jnp.zeros((pad,), jnp.int32)]).reshape(EROWS, 128)
    dst_r = jnp.concatenate(
        [edge_index[1], jnp.full((pad,), DUMMY, jnp.int32)]).reshape(EROWS, 128)
    batch_r = batch.reshape(GRID, 1, NBK)
    b1r = b1.reshape(1, H)
    b2r = b2.reshape(1, H)
    b3r = b3.reshape(1, H)
    blinr = blin.reshape(1, C)

    _deg_call, _agg_call = _sc_calls()
    dega, degb = _deg_call(dst_r)
    hs1 = _t1_call(x, W1, dega, degb)
    ag1 = _agg_call(*hs1, src_r, dst_r)
    hs2 = _mid_call(*ag1, *hs1, dega, degb, b1r, W2)
    ag2 = _agg_call(*hs2, src_r, dst_r)
    hs3 = _mid_call(*ag2, *hs2, dega, degb, b2r, W3)
    ag3 = _agg_call(*hs3, src_r, dst_r)
    return _t4_call(*ag3, *hs3, dega, degb, b3r, batch_r, Wlin, blinr)


# speedup vs baseline: 10.7727x; 1.1537x over previous
import functools

import jax
import jax.numpy as jnp
from jax import lax
from jax.experimental import pallas as pl
from jax.experimental.pallas import tpu as pltpu
from jax.experimental.pallas import tpu_sc as plsc

N = 100000
E = 1600000
F_IN = 33
H = 64
C = 2
G = 16

EROWS = 12800
EPAD = EROWS * 128
DUMMY = N
SLAB = 102400
NBK = 2000
GRID = N // NBK


def _fill(ref, value):
    @pl.loop(0, 128)
    def _(i):
        ref[i, :] = jnp.full((16,), value, jnp.float32)


def _zero_slab(slab, zv, s):
    @pl.loop(0, 50)
    def _(k):
        pltpu.sync_copy(zv, slab.at[pl.ds(s * 6400 + k * 128, 128)])


def _writeback(slab, out, s):
    @pl.when(s < 15)
    def _():
        row = s * 6256
        pltpu.sync_copy(slab.at[pl.ds(row, 6256)], out.at[pl.ds(row, 6256)])

    @pl.when(s == 15)
    def _():
        pltpu.sync_copy(slab.at[pl.ds(93840, 6160)], out.at[pl.ds(93840, 6160)])


def _deg_body(dst_hbm, dega, degb, slab, dbuf, ones_v, zv, sem):
    c = lax.axis_index("c")
    s = lax.axis_index("s")
    _fill(ones_v, 1.0)
    _fill(zv, 0.0)
    _zero_slab(slab, zv, s)
    plsc.subcore_barrier()
    base = c * 6400 + s * 400

    @pl.loop(0, 50)
    def _(b):
        pltpu.sync_copy(dst_hbm.at[pl.ds(base + b * 8, 8)], dbuf)
        descs = [
            pltpu.async_copy(ones_v, slab.at[dbuf.at[j]], sem, add=True)
            for j in range(8)
        ]
        for d in descs:
            d.wait()

    plsc.subcore_barrier()

    @pl.when(c == 0)
    def _():
        _writeback(slab, dega, s)

    @pl.when(c == 1)
    def _():
        _writeback(slab, degb, s)


def _fire_gathers(hs, sbuf, rows, gsem):
    return [
        pltpu.async_copy(hs.at[sbuf.at[j]], rows.at[pl.ds(j * 128, 128)],
                         gsem)
        for j in range(4)
    ]


def _drain_gathers(hs, sbuf, rows, gsem):
    for j in range(4):
        pltpu.make_async_copy(hs.at[sbuf.at[j]],
                              rows.at[pl.ds(j * 128, 128)], gsem).wait()


def _fire_scatters(rows, dbuf, slab, ssem):
    return [
        pltpu.async_copy(rows.at[pl.ds(j * 128, 128)], slab.at[dbuf.at[j]],
                         ssem, add=True)
        for j in range(4)
    ]


def _drain_scatters(rows, dbuf, slab, ssem):
    for j in range(4):
        pltpu.make_async_copy(rows.at[pl.ds(j * 128, 128)],
                              slab.at[dbuf.at[j]], ssem).wait()


def _one_pass(hs, agg, src_hbm, dst_hbm, s, slab, sbufs, dbufs, rowss, zv,
              gsem, ssem, isem):
    _zero_slab(slab, zv, s)
    plsc.subcore_barrier()
    base = s * 800
    nblk = 200

    pltpu.sync_copy(src_hbm.at[pl.ds(base, 4)], sbufs[0])
    pltpu.sync_copy(dst_hbm.at[pl.ds(base, 4)], dbufs[0])
    _fire_gathers(hs, sbufs[0], rowss[0], gsem)
    pltpu.async_copy(src_hbm.at[pl.ds(base + 4, 4)], sbufs[1], isem)
    pltpu.async_copy(dst_hbm.at[pl.ds(base + 4, 4)], dbufs[1], isem)

    @pl.loop(0, nblk, step=2)
    def _(bb):
        for p in (0, 1):
            b = bb + p
            q = 1 - p
            _drain_gathers(hs, sbufs[p], rowss[p], gsem)
            _fire_scatters(rowss[p], dbufs[p], slab, ssem)

            @pl.when(b < nblk - 1)
            def _():
                pltpu.make_async_copy(src_hbm.at[pl.ds(base, 4)],
                                      sbufs[q], isem).wait()
                pltpu.make_async_copy(dst_hbm.at[pl.ds(base, 4)],
                                      dbufs[q], isem).wait()
                _fire_gathers(hs, sbufs[q], rowss[q], gsem)

            _drain_scatters(rowss[p], dbufs[p], slab, ssem)

            @pl.when(b < nblk - 2)
            def _():
                row2 = base + (b + 2) * 4
                pltpu.async_copy(src_hbm.at[pl.ds(row2, 4)], sbufs[p], isem)
                pltpu.async_copy(dst_hbm.at[pl.ds(row2, 4)], dbufs[p], isem)

    plsc.subcore_barrier()
    _writeback(slab, agg, s)


def _agg_body(h0, h1, h2, h3, src_hbm, dst_hbm, a0, a1, a2, a3,
              slab, sbuf0, sbuf1, dbuf0, dbuf1, rows0, rows1, zv,
              gsem, ssem, isem):
    c = lax.axis_index("c")
    s = lax.axis_index("s")
    _fill(zv, 0.0)
    hs_list = [h0, h1, h2, h3]
    out_list = [a0, a1, a2, a3]
    for q in range(4):
        @pl.when(c == q // 2)
        def _(q=q):
            _one_pass(hs_list[q], out_list[q], src_hbm, dst_hbm, s, slab,
                      (sbuf0, sbuf1), (dbuf0, dbuf1), (rows0, rows1), zv,
                      gsem, ssem, isem)


@functools.cache
def _sc_calls():
    mesh = plsc.VectorSubcoreMesh(core_axis_name="c", subcore_axis_name="s")
    deg_call = pl.kernel(
        _deg_body,
        out_type=(jax.ShapeDtypeStruct((N, 16), jnp.float32),
                  jax.ShapeDtypeStruct((N, 16), jnp.float32)),
        mesh=mesh,
        compiler_params=pltpu.CompilerParams(use_tc_tiling_on_sc=False),
        scratch_types=(
            pltpu.VMEM_SHARED((SLAB, 16), jnp.float32),
            pltpu.VMEM((8, 128), jnp.int32),
            pltpu.VMEM((128, 16), jnp.float32),
            pltpu.VMEM((128, 16), jnp.float32),
            pltpu.SemaphoreType.DMA,
        ),
    )
    agg_call = pl.kernel(
        _agg_body,
        out_type=tuple(jax.ShapeDtypeStruct((N, 16), jnp.float32)
                       for _ in range(4)),
        mesh=mesh,
        compiler_params=pltpu.CompilerParams(use_tc_tiling_on_sc=False),
        scratch_types=(
            pltpu.VMEM_SHARED((SLAB, 16), jnp.float32),
            pltpu.VMEM((4, 128), jnp.int32),
            pltpu.VMEM((4, 128), jnp.int32),
            pltpu.VMEM((4, 128), jnp.int32),
            pltpu.VMEM((4, 128), jnp.int32),
            pltpu.VMEM((512, 16), jnp.float32),
            pltpu.VMEM((512, 16), jnp.float32),
            pltpu.VMEM((128, 16), jnp.float32),
            pltpu.SemaphoreType.DMA,
            pltpu.SemaphoreType.DMA,
            pltpu.SemaphoreType.DMA,
        ),
    )
    return deg_call, agg_call


def _dinv_blk(da, db):
    return lax.rsqrt(1.0 + da[:, 0:1] + db[:, 0:1])


def _t1_body(x_ref, w_ref, da, db, o0, o1, o2, o3):
    dinv = _dinv_blk(da, db)
    t = jnp.dot(x_ref[...], w_ref[...],
                preferred_element_type=jnp.float32) * dinv
    o0[...] = t[:, 0:16]
    o1[...] = t[:, 16:32]
    o2[...] = t[:, 32:48]
    o3[...] = t[:, 48:64]


def _mid_body(a0, a1, a2, a3, h0, h1, h2, h3, da, db, b_ref, w_ref,
              o0, o1, o2, o3):
    dinv = _dinv_blk(da, db)
    agg = jnp.concatenate([a0[...], a1[...], a2[...], a3[...]], axis=1)
    hsb = jnp.concatenate([h0[...], h1[...], h2[...], h3[...]], axis=1)
    h = jnp.maximum(dinv * (agg + hsb) + b_ref[...], 0.0)
    t = jnp.dot(h, w_ref[...], preferred_element_type=jnp.float32) * dinv
    o0[...] = t[:, 0:16]
    o1[...] = t[:, 16:32]
    o2[...] = t[:, 32:48]
    o3[...] = t[:, 48:64]


def _t4_body(a0, a1, a2, a3, h0, h1, h2, h3, da, db, b_ref, batch_ref,
             wlin_ref, blin_ref, out_ref, pooled, cnt):
    i = pl.program_id(0)

    @pl.when(i == 0)
    def _():
        pooled[...] = jnp.zeros((G, H), jnp.float32)
        cnt[...] = jnp.zeros((G, 128), jnp.float32)

    dinv = _dinv_blk(da, db)
    agg = jnp.concatenate([a0[...], a1[...], a2[...], a3[...]], axis=1)
    hsb = jnp.concatenate([h0[...], h1[...], h2[...], h3[...]], axis=1)
    h3v = dinv * (agg + hsb) + b_ref[...]
    bb = batch_ref[0, 0, :]
    gi = lax.broadcasted_iota(jnp.int32, (G, NBK), 0)
    oh = (bb[None, :] == gi).astype(jnp.float32)
    pooled[...] += jnp.dot(oh, h3v, preferred_element_type=jnp.float32)
    cnt[...] = cnt[...] + jnp.sum(oh, axis=1, keepdims=True)

    @pl.when(i == GRID - 1)
    def _():
        p = pooled[...] / jnp.maximum(cnt[:, 0:1], 1.0)
        out_ref[...] = (jnp.dot(p, wlin_ref[...],
                                preferred_element_type=jnp.float32)
                        + blin_ref[...])


def _row_spec(w):
    return pl.BlockSpec((NBK, w), lambda i: (i, 0))


def _full_spec(shape):
    nd = len(shape)
    return pl.BlockSpec(shape, lambda i: (0,) * nd)


_t1_call = pl.pallas_call(
    _t1_body,
    grid=(GRID,),
    in_specs=[
        _row_spec(F_IN),
        _full_spec((F_IN, H)),
        _row_spec(16),
        _row_spec(16),
    ],
    out_specs=[_row_spec(16)] * 4,
    out_shape=[jax.ShapeDtypeStruct((N, 16), jnp.float32)] * 4,
)

_mid_call = pl.pallas_call(
    _mid_body,
    grid=(GRID,),
    in_specs=[_row_spec(16)] * 8 + [
        _row_spec(16),
        _row_spec(16),
        _full_spec((1, H)),
        _full_spec((H, H)),
    ],
    out_specs=[_row_spec(16)] * 4,
    out_shape=[jax.ShapeDtypeStruct((N, 16), jnp.float32)] * 4,
)

_t4_call = pl.pallas_call(
    _t4_body,
    grid=(GRID,),
    in_specs=[_row_spec(16)] * 8 + [
        _row_spec(16),
        _row_spec(16),
        _full_spec((1, H)),
        pl.BlockSpec((1, 1, NBK), lambda i: (i, 0, 0)),
        _full_spec((H, C)),
        _full_spec((1, C)),
    ],
    out_specs=pl.BlockSpec((G, C), lambda i: (0, 0)),
    out_shape=jax.ShapeDtypeStruct((G, C), jnp.float32),
    scratch_shapes=[
        pltpu.VMEM((G, H), jnp.float32),
        pltpu.VMEM((G, 128), jnp.float32),
    ],
)


@jax.jit
def kernel(x, edge_index, batch, W1, b1, W2, b2, W3, b3, Wlin, blin):
    pad = EPAD - E
    src_r = jnp.concatenate(
        [edge_index[0], jnp.zeros((pad,), jnp.int32)]).reshape(EROWS, 128)
    dst_r = jnp.concatenate(
        [edge_index[1], jnp.full((pad,), DUMMY, jnp.int32)]).reshape(EROWS, 128)
    batch_r = batch.reshape(GRID, 1, NBK)
    b1r = b1.reshape(1, H)
    b2r = b2.reshape(1, H)
    b3r = b3.reshape(1, H)
    blinr = blin.reshape(1, C)

    _deg_call, _agg_call = _sc_calls()
    dega, degb = _deg_call(dst_r)
    hs1 = _t1_call(x, W1, dega, degb)
    ag1 = _agg_call(*hs1, src_r, dst_r)
    hs2 = _mid_call(*ag1, *hs1, dega, degb, b1r, W2)
    ag2 = _agg_call(*hs2, src_r, dst_r)
    hs3 = _mid_call(*ag2, *hs2, dega, degb, b2r, W3)
    ag3 = _agg_call(*hs3, src_r, dst_r)
    return _t4_call(*ag3, *hs3, dega, degb, b3r, batch_r, Wlin, blinr)


# speedup vs baseline: 10.7819x; 1.0009x over previous
import functools

import jax
import jax.numpy as jnp
from jax import lax
from jax.experimental import pallas as pl
from jax.experimental.pallas import tpu as pltpu
from jax.experimental.pallas import tpu_sc as plsc

N = 100000
E = 1600000
F_IN = 33
H = 64
C = 2
G = 16

EROWS = 12800
EPAD = EROWS * 128
DUMMY = N
SLAB = 102400
NBK = 2000
GRID = N // NBK


def _fill(ref, value):
    @pl.loop(0, 128)
    def _(i):
        ref[i, :] = jnp.full((16,), value, jnp.float32)


def _zero_slab(slab, zv, s):
    @pl.loop(0, 50)
    def _(k):
        pltpu.sync_copy(zv, slab.at[pl.ds(s * 6400 + k * 128, 128)])


def _writeback(slab, out, s):
    @pl.when(s < 15)
    def _():
        row = s * 6256
        pltpu.sync_copy(slab.at[pl.ds(row, 6256)], out.at[pl.ds(row, 6256)])

    @pl.when(s == 15)
    def _():
        pltpu.sync_copy(slab.at[pl.ds(93840, 6160)], out.at[pl.ds(93840, 6160)])


def _deg_body(dst_hbm, dega, degb, slab, dbuf, ones_v, zv, sem):
    c = lax.axis_index("c")
    s = lax.axis_index("s")
    _fill(ones_v, 1.0)
    _fill(zv, 0.0)
    _zero_slab(slab, zv, s)
    plsc.subcore_barrier()
    base = c * 6400 + s * 400

    @pl.loop(0, 50)
    def _(b):
        pltpu.sync_copy(dst_hbm.at[pl.ds(base + b * 8, 8)], dbuf)
        descs = [
            pltpu.async_copy(ones_v, slab.at[dbuf.at[j]], sem, add=True)
            for j in range(8)
        ]
        for d in descs:
            d.wait()

    plsc.subcore_barrier()

    @pl.when(c == 0)
    def _():
        _writeback(slab, dega, s)

    @pl.when(c == 1)
    def _():
        _writeback(slab, degb, s)


def _fire_gathers(hs, sbuf, rows, gsem):
    return [
        pltpu.async_copy(hs.at[sbuf.at[j]], rows.at[pl.ds(j * 128, 128)],
                         gsem)
        for j in range(4)
    ]


def _drain_gathers(hs, sbuf, rows, gsem):
    pltpu.make_async_copy(hs.at[pl.ds(0, 512)], rows, gsem).wait()


def _fire_scatters(rows, dbuf, slab, ssem):
    return [
        pltpu.async_copy(rows.at[pl.ds(j * 128, 128)], slab.at[dbuf.at[j]],
                         ssem, add=True)
        for j in range(4)
    ]


def _drain_scatters(rows, slab, ssem):
    pltpu.make_async_copy(slab.at[pl.ds(0, 512)], rows, ssem).wait()


def _one_pass(hs, agg, src_hbm, dst_hbm, s, slab, sbufs, dbufs, rowss, zv,
              gsem, ssem, isem):
    _zero_slab(slab, zv, s)
    plsc.subcore_barrier()
    base = s * 800
    nblk = 200

    pltpu.sync_copy(src_hbm.at[pl.ds(base, 4)], sbufs[0])
    pltpu.sync_copy(dst_hbm.at[pl.ds(base, 4)], dbufs[0])
    _fire_gathers(hs, sbufs[0], rowss[0], gsem)
    pltpu.async_copy(src_hbm.at[pl.ds(base + 4, 4)], sbufs[1], isem)
    pltpu.async_copy(dst_hbm.at[pl.ds(base + 4, 4)], dbufs[1], isem)

    @pl.loop(0, nblk, step=2)
    def _(bb):
        for p in (0, 1):
            b = bb + p
            q = 1 - p
            _drain_gathers(hs, sbufs[p], rowss[p], gsem)
            _fire_scatters(rowss[p], dbufs[p], slab, ssem)

            @pl.when(b < nblk - 1)
            def _():
                pltpu.make_async_copy(src_hbm.at[pl.ds(base, 4)],
                                      sbufs[q], isem).wait()
                pltpu.make_async_copy(dst_hbm.at[pl.ds(base, 4)],
                                      dbufs[q], isem).wait()
                _fire_gathers(hs, sbufs[q], rowss[q], gsem)

            _drain_scatters(rowss[p], slab, ssem)

            @pl.when(b < nblk - 2)
            def _():
                row2 = base + (b + 2) * 4
                pltpu.async_copy(src_hbm.at[pl.ds(row2, 4)], sbufs[p], isem)
                pltpu.async_copy(dst_hbm.at[pl.ds(row2, 4)], dbufs[p], isem)

    plsc.subcore_barrier()
    _writeback(slab, agg, s)


def _agg_body(h0, h1, h2, h3, src_hbm, dst_hbm, a0, a1, a2, a3,
              slab, sbuf0, sbuf1, dbuf0, dbuf1, rows0, rows1, zv,
              gsem, ssem, isem):
    c = lax.axis_index("c")
    s = lax.axis_index("s")
    _fill(zv, 0.0)
    hs_list = [h0, h1, h2, h3]
    out_list = [a0, a1, a2, a3]
    for q in range(4):
        @pl.when(c == q // 2)
        def _(q=q):
            _one_pass(hs_list[q], out_list[q], src_hbm, dst_hbm, s, slab,
                      (sbuf0, sbuf1), (dbuf0, dbuf1), (rows0, rows1), zv,
                      gsem, ssem, isem)


@functools.cache
def _sc_calls():
    mesh = plsc.VectorSubcoreMesh(core_axis_name="c", subcore_axis_name="s")
    deg_call = pl.kernel(
        _deg_body,
        out_type=(jax.ShapeDtypeStruct((N, 16), jnp.float32),
                  jax.ShapeDtypeStruct((N, 16), jnp.float32)),
        mesh=mesh,
        compiler_params=pltpu.CompilerParams(use_tc_tiling_on_sc=False),
        scratch_types=(
            pltpu.VMEM_SHARED((SLAB, 16), jnp.float32),
            pltpu.VMEM((8, 128), jnp.int32),
            pltpu.VMEM((128, 16), jnp.float32),
            pltpu.VMEM((128, 16), jnp.float32),
            pltpu.SemaphoreType.DMA,
        ),
    )
    agg_call = pl.kernel(
        _agg_body,
        out_type=tuple(jax.ShapeDtypeStruct((N, 16), jnp.float32)
                       for _ in range(4)),
        mesh=mesh,
        compiler_params=pltpu.CompilerParams(use_tc_tiling_on_sc=False),
        scratch_types=(
            pltpu.VMEM_SHARED((SLAB, 16), jnp.float32),
            pltpu.VMEM((4, 128), jnp.int32),
            pltpu.VMEM((4, 128), jnp.int32),
            pltpu.VMEM((4, 128), jnp.int32),
            pltpu.VMEM((4, 128), jnp.int32),
            pltpu.VMEM((512, 16), jnp.float32),
            pltpu.VMEM((512, 16), jnp.float32),
            pltpu.VMEM((128, 16), jnp.float32),
            pltpu.SemaphoreType.DMA,
            pltpu.SemaphoreType.DMA,
            pltpu.SemaphoreType.DMA,
        ),
    )
    return deg_call, agg_call


def _dinv_blk(da, db):
    return lax.rsqrt(1.0 + da[:, 0:1] + db[:, 0:1])


def _t1_body(x_ref, w_ref, da, db, o0, o1, o2, o3):
    dinv = _dinv_blk(da, db)
    t = jnp.dot(x_ref[...], w_ref[...],
                preferred_element_type=jnp.float32) * dinv
    o0[...] = t[:, 0:16]
    o1[...] = t[:, 16:32]
    o2[...] = t[:, 32:48]
    o3[...] = t[:, 48:64]


def _mid_body(a0, a1, a2, a3, h0, h1, h2, h3, da, db, b_ref, w_ref,
              o0, o1, o2, o3):
    dinv = _dinv_blk(da, db)
    agg = jnp.concatenate([a0[...], a1[...], a2[...], a3[...]], axis=1)
    hsb = jnp.concatenate([h0[...], h1[...], h2[...], h3[...]], axis=1)
    h = jnp.maximum(dinv * (agg + hsb) + b_ref[...], 0.0)
    t = jnp.dot(h, w_ref[...], preferred_element_type=jnp.float32) * dinv
    o0[...] = t[:, 0:16]
    o1[...] = t[:, 16:32]
    o2[...] = t[:, 32:48]
    o3[...] = t[:, 48:64]


def _t4_body(a0, a1, a2, a3, h0, h1, h2, h3, da, db, b_ref, batch_ref,
             wlin_ref, blin_ref, out_ref, pooled, cnt):
    i = pl.program_id(0)

    @pl.when(i == 0)
    def _():
        pooled[...] = jnp.zeros((G, H), jnp.float32)
        cnt[...] = jnp.zeros((G, 128), jnp.float32)

    dinv = _dinv_blk(da, db)
    agg = jnp.concatenate([a0[...], a1[...], a2[...], a3[...]], axis=1)
    hsb = jnp.concatenate([h0[...], h1[...], h2[...], h3[...]], axis=1)
    h3v = dinv * (agg + hsb) + b_ref[...]
    bb = batch_ref[0, 0, :]
    gi = lax.broadcasted_iota(jnp.int32, (G, NBK), 0)
    oh = (bb[None, :] == gi).astype(jnp.float32)
    pooled[...] += jnp.dot(oh, h3v, preferred_element_type=jnp.float32)
    cnt[...] = cnt[...] + jnp.sum(oh, axis=1, keepdims=True)

    @pl.when(i == GRID - 1)
    def _():
        p = pooled[...] / jnp.maximum(cnt[:, 0:1], 1.0)
        out_ref[...] = (jnp.dot(p, wlin_ref[...],
                                preferred_element_type=jnp.float32)
                        + blin_ref[...])


def _row_spec(w):
    return pl.BlockSpec((NBK, w), lambda i: (i, 0))


def _full_spec(shape):
    nd = len(shape)
    return pl.BlockSpec(shape, lambda i: (0,) * nd)


_t1_call = pl.pallas_call(
    _t1_body,
    grid=(GRID,),
    in_specs=[
        _row_spec(F_IN),
        _full_spec((F_IN, H)),
        _row_spec(16),
        _row_spec(16),
    ],
    out_specs=[_row_spec(16)] * 4,
    out_shape=[jax.ShapeDtypeStruct((N, 16), jnp.float32)] * 4,
)

_mid_call = pl.pallas_call(
    _mid_body,
    grid=(GRID,),
    in_specs=[_row_spec(16)] * 8 + [
        _row_spec(16),
        _row_spec(16),
        _full_spec((1, H)),
        _full_spec((H, H)),
    ],
    out_specs=[_row_spec(16)] * 4,
    out_shape=[jax.ShapeDtypeStruct((N, 16), jnp.float32)] * 4,
)

_t4_call = pl.pallas_call(
    _t4_body,
    grid=(GRID,),
    in_specs=[_row_spec(16)] * 8 + [
        _row_spec(16),
        _row_spec(16),
        _full_spec((1, H)),
        pl.BlockSpec((1, 1, NBK), lambda i: (i, 0, 0)),
        _full_spec((H, C)),
        _full_spec((1, C)),
    ],
    out_specs=pl.BlockSpec((G, C), lambda i: (0, 0)),
    out_shape=jax.ShapeDtypeStruct((G, C), jnp.float32),
    scratch_shapes=[
        pltpu.VMEM((G, H), jnp.float32),
        pltpu.VMEM((G, 128), jnp.float32),
    ],
)


@jax.jit
def kernel(x, edge_index, batch, W1, b1, W2, b2, W3, b3, Wlin, blin):
    pad = EPAD - E
    src_r = jnp.concatenate(
        [edge_index[0], jnp.zeros((pad,), jnp.int32)]).reshape(EROWS, 128)
    dst_r = jnp.concatenate(
        [edge_index[1], jnp.full((pad,), DUMMY, jnp.int32)]).reshape(EROWS, 128)
    batch_r = batch.reshape(GRID, 1, NBK)
    b1r = b1.reshape(1, H)
    b2r = b2.reshape(1, H)
    b3r = b3.reshape(1, H)
    blinr = blin.reshape(1, C)

    _deg_call, _agg_call = _sc_calls()
    dega, degb = _deg_call(dst_r)
    hs1 = _t1_call(x, W1, dega, degb)
    ag1 = _agg_call(*hs1, src_r, dst_r)
    hs2 = _mid_call(*ag1, *hs1, dega, degb, b1r, W2)
    ag2 = _agg_call(*hs2, src_r, dst_r)
    hs3 = _mid_call(*ag2, *hs2, dega, degb, b2r, W3)
    ag3 = _agg_call(*hs3, src_r, dst_r)
    return _t4_call(*ag3, *hs3, dega, degb, b3r, batch_r, Wlin, blinr)
